# baseline (speedup 1.0000x reference)
import functools
import jax
import jax.numpy as jnp
from jax import lax
from jax.experimental import pallas as pl
from jax.experimental.pallas import tpu as pltpu
from jax.experimental.pallas import tpu_sc as plsc

N_IN = 162
N_OUT = 42
NEIGH = 7
LANES = 16
NUM_CORES = 2
NUM_SUBCORES = 16
NUM_WORKERS = NUM_CORES * NUM_SUBCORES
CHUNK = 256


def _sc_pool(total_rows):
    rows_per_worker = total_rows // NUM_WORKERS
    n_chunks = rows_per_worker // CHUNK
    mesh = plsc.VectorSubcoreMesh(
        core_axis_name="c", subcore_axis_name="s",
        num_cores=NUM_CORES, num_subcores=NUM_SUBCORES)

    @functools.partial(
        pl.kernel,
        out_type=jax.ShapeDtypeStruct((total_rows, N_OUT), jnp.float32),
        mesh=mesh,
        scratch_types=[
            pltpu.VMEM((CHUNK, N_IN), jnp.float32),
            pltpu.VMEM((CHUNK, N_OUT), jnp.float32),
            pltpu.VMEM((N_OUT * NEIGH * LANES,), jnp.int32),
        ],
        compiler_params=pltpu.CompilerParams(needs_layout_passes=False),
    )
    def run(x_hbm, gidx_hbm, out_hbm, in_v, out_v, gidx_v):
        wid = lax.axis_index("s") * NUM_CORES + lax.axis_index("c")
        base_row = wid * rows_per_worker
        pltpu.sync_copy(gidx_hbm, gidx_v)
        riota = lax.iota(jnp.int32, LANES)

        def chunk_body(g, _):
            row0 = base_row + g * CHUNK
            pltpu.sync_copy(x_hbm.at[pl.ds(row0, CHUNK)], in_v)
            for v in range(N_OUT):
                cvecs = [gidx_v[pl.ds((v * NEIGH + j) * LANES, LANES)]
                         for j in range(NEIGH)]
                colv = jnp.full((LANES,), v, dtype=jnp.int32)

                @plsc.parallel_loop(0, CHUNK // LANES, 1, unroll=2)
                def row_body(r, cvecs=cvecs, colv=colv):
                    rvec = riota + r * LANES
                    g0 = plsc.load_gather(in_v, [rvec, cvecs[0]])
                    g1 = plsc.load_gather(in_v, [rvec, cvecs[1]])
                    g2 = plsc.load_gather(in_v, [rvec, cvecs[2]])
                    g3 = plsc.load_gather(in_v, [rvec, cvecs[3]])
                    g4 = plsc.load_gather(in_v, [rvec, cvecs[4]])
                    g5 = plsc.load_gather(in_v, [rvec, cvecs[5]])
                    g6 = plsc.load_gather(in_v, [rvec, cvecs[6]])
                    s = ((g0 + g1) + (g2 + g3)) + ((g4 + g5) + g6)
                    acc = s * jnp.float32(1.0 / NEIGH)
                    plsc.store_scatter(out_v, [rvec, colv], acc)

            pltpu.sync_copy(out_v, out_hbm.at[pl.ds(row0, CHUNK)])
            return 0

        lax.fori_loop(0, n_chunks, chunk_body, 0)

    return run


def kernel(x, down_neigh_indices):
    b, c, n_in = x.shape
    total_rows = b * c
    xf = x.reshape(total_rows, n_in)
    flat_idx = down_neigh_indices.reshape(-1).astype(jnp.int32)
    gidx = jnp.broadcast_to(flat_idx[:, None],
                            (N_OUT * NEIGH, LANES)).reshape(-1)
    out = _sc_pool(total_rows)(xf, gidx)
    return out.reshape(b, c, N_OUT)


# speedup vs baseline: 3.6598x; 3.6598x over previous
import jax
import jax.numpy as jnp
from jax import lax
from jax.experimental import pallas as pl
from jax.experimental.pallas import tpu as pltpu

N_IN = 162
N_OUT = 42
NEIGH = 7
U = 4
CB = 512
NCB = 2048 // CB


def _body(uv_ref, uw_ref, x_ref, o_ref):
    k = pl.program_id(1)

    @pl.when(k == 0)
    def _():
        o_ref[...] = jnp.zeros_like(o_ref)

    xb = x_ref[0]
    for u in range(U):
        v = uv_ref[k, u]
        w = uw_ref[k, u]
        o_ref[pl.ds(v, 1)] += (w * xb)[None]


def _plane_pool(batch, chans):
    return pl.pallas_call(
        _body,
        grid_spec=pltpu.PrefetchScalarGridSpec(
            num_scalar_prefetch=2,
            grid=(NCB, N_IN),
            in_specs=[
                pl.BlockSpec((1, batch, CB), lambda cb, k, uv, uw: (k, 0, cb)),
            ],
            out_specs=pl.BlockSpec((N_OUT, batch, CB),
                                   lambda cb, k, uv, uw: (0, 0, cb)),
        ),
        out_shape=jax.ShapeDtypeStruct((N_OUT, batch, chans), jnp.float32),
    )


def kernel(x, down_neigh_indices):
    b, c, n_in = x.shape
    idx32 = down_neigh_indices.astype(jnp.int32)
    onehot = jax.nn.one_hot(idx32, n_in, axis=-1, dtype=jnp.float32)
    m = onehot.sum(1).T * jnp.float32(1.0 / NEIGH)
    uw, uv = lax.top_k(m, U)
    xt = jnp.transpose(x, (2, 0, 1))
    out_t = _plane_pool(b, c)(uv.astype(jnp.int32), uw, xt)
    return jnp.transpose(out_t, (1, 2, 0))
